# baseline (speedup 1.0000x reference)
import functools

import jax
import jax.numpy as jnp
from jax import lax
from jax.experimental import pallas as pl
from jax.experimental.pallas import tpu as pltpu
from jax.experimental.pallas import tpu_sc as plsc

NC = 2
NS = 16
NW = NC * NS
LANES = 16
CH = 80


def _make_sc_spmm(n, e, d):
    assert e % NW == 0
    epw = e // NW
    assert epw % CH == 0
    nit = epw // CH
    zr = 128
    np_ = -(-n // (NS * zr)) * (NS * zr)
    rpt = np_ // NS
    nzc = rpt // zr
    nvec = d // LANES

    mesh = plsc.VectorSubcoreMesh(
        core_axis_name="c", subcore_axis_name="s",
        num_cores=NC, num_subcores=NS)

    @functools.partial(
        pl.kernel,
        out_type=jax.ShapeDtypeStruct((2 * np_, d), jnp.float32),
        mesh=mesh,
        scratch_types=[
            pltpu.VMEM((CH,), jnp.int32),
            pltpu.VMEM((CH,), jnp.int32),
            pltpu.VMEM((CH,), jnp.float32),
            pltpu.VMEM((CH, d), jnp.float32),
            pltpu.VMEM((zr, d), jnp.float32),
            pltpu.VMEM_SHARED((np_, d), jnp.float32),
            pltpu.SemaphoreType.DMA,
        ],
    )
    def spmm(feat_hbm, src_hbm, dst_hbm, ew_hbm, out_hbm,
             src_v, dst_v, w_v, rows_v, z_v, acc_sh, sem):
        c = lax.axis_index("c")
        s = lax.axis_index("s")
        wid = c * NS + s

        zeros = jnp.zeros((LANES,), jnp.float32)

        def zero_row(r, carry):
            for j in range(nvec):
                z_v[r, pl.ds(j * LANES, LANES)] = zeros
            return carry

        lax.fori_loop(0, zr, zero_row, 0)
        for k in range(nzc):
            pltpu.sync_copy(z_v, acc_sh.at[pl.ds(s * rpt + k * zr, zr)])
        plsc.subcore_barrier()

        ebase = wid * epw

        def chunk(i, carry):
            off = ebase + i * CH
            pltpu.sync_copy(src_hbm.at[pl.ds(off, CH)], src_v)
            pltpu.sync_copy(dst_hbm.at[pl.ds(off, CH)], dst_v)
            pltpu.sync_copy(ew_hbm.at[pl.ds(off, CH)], w_v)
            pltpu.async_copy(feat_hbm.at[src_v], rows_v, sem).wait()

            def scale16(g, c2):
                wvec = w_v[pl.ds(g * LANES, LANES)]
                e0 = g * LANES
                for l in range(LANES):
                    wl = wvec.at[jnp.full((LANES,), l, jnp.int32)].get(
                        mode="promise_in_bounds")
                    for j in range(nvec):
                        sl = pl.ds(j * LANES, LANES)
                        rows_v[e0 + l, sl] = rows_v[e0 + l, sl] * wl
                return c2

            lax.fori_loop(0, CH // LANES, scale16, 0)
            pltpu.sync_copy(rows_v, acc_sh.at[dst_v], add=True)
            return carry

        lax.fori_loop(0, nit, chunk, 0)
        plsc.subcore_barrier()

        obase = c * np_ + s * rpt
        for k in range(nzc):
            pltpu.sync_copy(acc_sh.at[pl.ds(s * rpt + k * zr, zr)],
                            out_hbm.at[pl.ds(obase + k * zr, zr)])

    return spmm, np_


def _tc_matmul_body(p0_ref, p1_ref, w_ref, b_ref, o_ref):
    acc = p0_ref[...] + p1_ref[...]
    o_ref[...] = (
        jnp.dot(acc, w_ref[...], preferred_element_type=jnp.float32)
        + b_ref[...]
    )


def _make_tc_matmul(n, d_in, d_out, bm):
    grid = (n // bm,)
    return pl.pallas_call(
        _tc_matmul_body,
        grid=grid,
        in_specs=[
            pl.BlockSpec((bm, d_in), lambda i: (i, 0)),
            pl.BlockSpec((bm, d_in), lambda i: (i, 0)),
            pl.BlockSpec((d_in, d_out), lambda i: (0, 0)),
            pl.BlockSpec((1, d_out), lambda i: (0, 0)),
        ],
        out_specs=pl.BlockSpec((bm, d_out), lambda i: (i, 0)),
        out_shape=jax.ShapeDtypeStruct((n, d_out), jnp.float32),
    )


def kernel(features, edge_index, edge_weight, W, bias):
    n, d_in = features.shape
    d_out = W.shape[1]
    e = edge_weight.shape[0]
    src = edge_index[0].astype(jnp.int32)
    dst = edge_index[1].astype(jnp.int32)
    ew = edge_weight.astype(jnp.float32)

    spmm, np_ = _make_sc_spmm(n, e, d_in)
    partials = spmm(features, src, dst, ew)
    p0 = partials[:n]
    p1 = partials[np_:np_ + n]
    out = _make_tc_matmul(n, d_in, d_out, 1000)(
        p0, p1, W, bias.reshape(1, d_out))
    return out


# speedup vs baseline: 2.3313x; 2.3313x over previous
import functools

import jax
import jax.numpy as jnp
from jax import lax
from jax.experimental import pallas as pl
from jax.experimental.pallas import tpu as pltpu
from jax.experimental.pallas import tpu_sc as plsc

NC = 2
NS = 16
NW = NC * NS
LANES = 16
CH = 80


def _make_sc_spmm(n, e, d):
    assert e % NW == 0
    epw = e // NW
    assert epw % CH == 0
    nit = epw // CH
    np_ = -(-n // (NS * 8)) * (NS * 8)
    rpt = np_ // NS
    nvec = d // LANES

    mesh = plsc.VectorSubcoreMesh(
        core_axis_name="c", subcore_axis_name="s",
        num_cores=NC, num_subcores=NS)

    @functools.partial(
        pl.kernel,
        out_type=jax.ShapeDtypeStruct((2 * np_, d), jnp.float32),
        mesh=mesh,
        scratch_types=[
            pltpu.VMEM((nit, CH), jnp.int32),
            [pltpu.VMEM((CH,), jnp.int32) for _ in range(2)],
            [pltpu.VMEM((CH,), jnp.float32) for _ in range(2)],
            [pltpu.VMEM((CH, d), jnp.float32) for _ in range(2)],
            pltpu.VMEM_SHARED((np_, d), jnp.float32),
            [pltpu.SemaphoreType.DMA for _ in range(2)],
            [pltpu.SemaphoreType.DMA for _ in range(2)],
            [pltpu.SemaphoreType.DMA for _ in range(2)],
        ],
    )
    def spmm(feat_hbm, src_hbm, dst_hbm, ew_hbm, out_hbm,
             dsts_v, srcs, ws, rows, acc_sh, isem, gsem, ssem):
        c = lax.axis_index("c")
        s = lax.axis_index("s")
        wid = c * NS + s
        ebase = wid * epw

        zeros = jnp.zeros((LANES,), jnp.float32)

        def zero_row(r, carry):
            for j in range(nvec):
                rows[0][r, pl.ds(j * LANES, LANES)] = zeros
            return carry

        lax.fori_loop(0, CH, zero_row, 0)
        zoff = 0
        while zoff < rpt:
            zn = min(CH, rpt - zoff)
            pltpu.sync_copy(rows[0].at[pl.ds(0, zn)],
                            acc_sh.at[pl.ds(s * rpt + zoff, zn)])
            zoff += zn

        pltpu.sync_copy(dst_hbm.at[wid], dsts_v)
        plsc.subcore_barrier()

        def idx_start(i, b):
            pltpu.make_async_copy(
                src_hbm.at[pl.ds(ebase + i * CH, CH)], srcs[b],
                isem[b]).start()
            pltpu.make_async_copy(
                ew_hbm.at[pl.ds(ebase + i * CH, CH)], ws[b],
                isem[b]).start()

        def idx_wait(i, b):
            pltpu.make_async_copy(
                src_hbm.at[pl.ds(ebase + i * CH, CH)], srcs[b],
                isem[b]).wait()
            pltpu.make_async_copy(
                ew_hbm.at[pl.ds(ebase + i * CH, CH)], ws[b],
                isem[b]).wait()

        def gather_start(i, b):
            pltpu.make_async_copy(
                feat_hbm.at[srcs[b]], rows[b], gsem[b]).start()

        def gather_wait(i, b):
            pltpu.make_async_copy(
                feat_hbm.at[srcs[b]], rows[b], gsem[b]).wait()

        def scatter_start(i, b):
            pltpu.make_async_copy(
                rows[b], acc_sh.at[dsts_v.at[i]], ssem[b]).start(add=True)

        def scatter_wait(i, b):
            pltpu.make_async_copy(
                rows[b], acc_sh.at[dsts_v.at[i]], ssem[b]).wait()

        def scale(i, b):
            for g in range(CH // LANES):
                wvec = ws[b][pl.ds(g * LANES, LANES)]
                e0 = g * LANES
                for l in range(LANES):
                    wl = wvec.at[jnp.full((LANES,), l, jnp.int32)].get(
                        mode="promise_in_bounds")
                    for j in range(nvec):
                        sl = pl.ds(j * LANES, LANES)
                        rows[b][e0 + l, sl] = rows[b][e0 + l, sl] * wl

        idx_start(0, 0)
        idx_start(1, 1)
        idx_wait(0, 0)
        gather_start(0, 0)

        def step(i, b):
            nb = 1 - b

            @pl.when(jnp.logical_and(i >= 1, i + 1 < nit))
            def _free_rows():
                scatter_wait(i - 1, nb)

            @pl.when(i + 1 < nit)
            def _next_gather():
                idx_wait(i + 1, nb)
                gather_start(i + 1, nb)

            gather_wait(i, b)
            scale(i, b)

            @pl.when(i + 2 < nit)
            def _prefetch_idx():
                idx_start(i + 2, b)

            scatter_start(i, b)

        def outer(i0, carry):
            step(2 * i0, 0)
            step(2 * i0 + 1, 1)
            return carry

        lax.fori_loop(0, nit // 2, outer, 0)
        for i in range(2 * (nit // 2), nit):
            step(i, i % 2)
        scatter_wait(nit - 2, (nit - 2) % 2)
        scatter_wait(nit - 1, (nit - 1) % 2)
        plsc.subcore_barrier()

        obase = c * np_ + s * rpt
        woff = 0
        while woff < rpt:
            wn = min(CH, rpt - woff)
            pltpu.sync_copy(acc_sh.at[pl.ds(s * rpt + woff, wn)],
                            out_hbm.at[pl.ds(obase + woff, wn)])
            woff += wn

    return spmm, np_


def _tc_matmul_body(p0_ref, p1_ref, w_ref, b_ref, o_ref):
    acc = p0_ref[...] + p1_ref[...]
    o_ref[...] = (
        jnp.dot(acc, w_ref[...], preferred_element_type=jnp.float32)
        + b_ref[...]
    )


def _make_tc_matmul(n, d_in, d_out, bm):
    grid = (n // bm,)
    return pl.pallas_call(
        _tc_matmul_body,
        grid=grid,
        in_specs=[
            pl.BlockSpec((bm, d_in), lambda i: (i, 0)),
            pl.BlockSpec((bm, d_in), lambda i: (i, 0)),
            pl.BlockSpec((d_in, d_out), lambda i: (0, 0)),
            pl.BlockSpec((1, d_out), lambda i: (0, 0)),
        ],
        out_specs=pl.BlockSpec((bm, d_out), lambda i: (i, 0)),
        out_shape=jax.ShapeDtypeStruct((n, d_out), jnp.float32),
    )


def kernel(features, edge_index, edge_weight, W, bias):
    n, d_in = features.shape
    d_out = W.shape[1]
    e = edge_weight.shape[0]
    epw = e // NW
    nit = epw // CH
    src = edge_index[0].astype(jnp.int32)
    dst = edge_index[1].astype(jnp.int32).reshape(NW, nit, CH)
    ew = edge_weight.astype(jnp.float32)

    spmm, np_ = _make_sc_spmm(n, e, d_in)
    partials = spmm(features, src, dst, ew)
    p0 = partials[:n]
    p1 = partials[np_:np_ + n]
    out = _make_tc_matmul(n, d_in, d_out, 1000)(
        p0, p1, W, bias.reshape(1, d_out))
    return out
